# initial kernel scaffold (unmeasured)
import jax
import jax.numpy as jnp
from jax import lax
from jax.experimental import pallas as pl
from jax.experimental.pallas import tpu as pltpu

N_DEV = 4


def kernel(partial, resid, gamma):
    _, m, d = partial.shape
    gamma2d = gamma.reshape(1, d)

    def body(p_ref, r_ref, g_ref, out_ref, comm_ref, send_sems, recv_sems):
        my_x = lax.axis_index("x")
        my_y = lax.axis_index("y")
        my_z = lax.axis_index("z")
        left = (my_y - 1) % N_DEV
        right = (my_y + 1) % N_DEV

        barrier_sem = pltpu.get_barrier_semaphore()
        for nbr in [left, right]:
            pl.semaphore_signal(
                barrier_sem, inc=1,
                device_id=(my_x, nbr, my_z),
                device_id_type=pl.DeviceIdType.MESH,
            )
        pl.semaphore_wait(barrier_sem, 2)

        mine = p_ref[0]
        out_ref[...] = mine + r_ref[...]
        comm_ref[0] = mine.astype(jnp.bfloat16)

        for h in range(N_DEV - 1):
            send_slot = h % 2
            recv_slot = (h + 1) % 2
            rdma = pltpu.make_async_remote_copy(
                src_ref=comm_ref.at[send_slot],
                dst_ref=comm_ref.at[recv_slot],
                send_sem=send_sems.at[send_slot],
                recv_sem=recv_sems.at[recv_slot],
                device_id=(my_x, right, my_z),
                device_id_type=pl.DeviceIdType.MESH,
            )
            rdma.start()
            rdma.wait()
            out_ref[...] += comm_ref[recv_slot].astype(jnp.float32)

        y = out_ref[...]
        ms = jnp.mean(y * y, axis=-1, keepdims=True)
        out_ref[...] = y * lax.rsqrt(ms + 1e-6) * g_ref[...]

    return pl.pallas_call(
        body,
        out_shape=jax.ShapeDtypeStruct((m, d), jnp.float32),
        in_specs=[
            pl.BlockSpec(memory_space=pltpu.VMEM),
            pl.BlockSpec(memory_space=pltpu.VMEM),
            pl.BlockSpec(memory_space=pltpu.VMEM),
        ],
        out_specs=pl.BlockSpec(memory_space=pltpu.VMEM),
        scratch_shapes=[
            pltpu.VMEM((2, m, d), jnp.bfloat16),
            pltpu.SemaphoreType.DMA((2,)),
            pltpu.SemaphoreType.DMA((2,)),
        ],
        compiler_params=pltpu.CompilerParams(
            collective_id=0,
            vmem_limit_bytes=128 * 1024 * 1024,
        ),
    )(partial, resid, gamma2d)


# baseline (device time: 186031 ns/iter reference)
import jax
import jax.numpy as jnp
from jax import lax
from jax.experimental import pallas as pl
from jax.experimental.pallas import tpu as pltpu

N_DEV = 4


def kernel(partial, resid, gamma):
    _, m, d = partial.shape
    chunk = m // N_DEV
    partial_bf = partial.astype(jnp.bfloat16).reshape(N_DEV, chunk, d)
    gamma2d = gamma.reshape(1, d)

    def body(p_ref, r_ref, g_ref, out_ref,
             rs_recv, rs_send, res_bf, resid_vmem, local_sem,
             rs_ssems, rs_rsems, ag_ssems, ag_rsems):
        my_x = lax.axis_index("x")
        my_y = lax.axis_index("y")
        my_z = lax.axis_index("z")
        left = (my_y - 1) % N_DEV
        right = (my_y + 1) % N_DEV
        own = (my_y + 1) % N_DEV

        barrier_sem = pltpu.get_barrier_semaphore()
        for nbr in [left, right]:
            pl.semaphore_signal(
                barrier_sem, inc=1,
                device_id=(my_x, nbr, my_z),
                device_id_type=pl.DeviceIdType.MESH,
            )
        pl.semaphore_wait(barrier_sem, 2)

        resid_dma = pltpu.make_async_copy(
            r_ref.at[pl.ds(own * chunk, chunk), :], resid_vmem, local_sem)
        resid_dma.start()

        for s in range(N_DEV - 1):
            src = p_ref.at[my_y] if s == 0 else rs_send.at[s - 1]
            rdma = pltpu.make_async_remote_copy(
                src_ref=src,
                dst_ref=rs_recv.at[s % 2],
                send_sem=rs_ssems.at[s],
                recv_sem=rs_rsems.at[s],
                device_id=(my_x, right, my_z),
                device_id_type=pl.DeviceIdType.MESH,
            )
            rdma.start()
            rdma.wait()
            c = (my_y - 1 - s) % N_DEV
            acc = rs_recv[s % 2].astype(jnp.float32) + p_ref[c].astype(
                jnp.float32
            )
            if s < N_DEV - 2:
                rs_send[s] = acc.astype(jnp.bfloat16)
            else:
                resid_dma.wait()
                yv = acc + resid_vmem[...]
                ms = jnp.mean(yv * yv, axis=-1, keepdims=True)
                o = yv * lax.rsqrt(ms + 1e-6) * g_ref[...]
                out_ref[pl.ds(c * chunk, chunk), :] = o
                res_bf[c] = o.astype(jnp.bfloat16)

        for s in range(N_DEV - 1):
            sc = (my_y + 1 - s) % N_DEV
            rc = (my_y - s) % N_DEV
            rdma = pltpu.make_async_remote_copy(
                src_ref=res_bf.at[sc],
                dst_ref=res_bf.at[sc],
                send_sem=ag_ssems.at[s],
                recv_sem=ag_rsems.at[s],
                device_id=(my_x, right, my_z),
                device_id_type=pl.DeviceIdType.MESH,
            )
            rdma.start()
            rdma.wait()
            out_ref[pl.ds(rc * chunk, chunk), :] = res_bf[rc].astype(
                jnp.float32
            )

    return pl.pallas_call(
        body,
        out_shape=jax.ShapeDtypeStruct((m, d), jnp.float32),
        in_specs=[
            pl.BlockSpec(memory_space=pltpu.VMEM),
            pl.BlockSpec(memory_space=pltpu.MemorySpace.HBM),
            pl.BlockSpec(memory_space=pltpu.VMEM),
        ],
        out_specs=pl.BlockSpec(memory_space=pltpu.VMEM),
        scratch_shapes=[
            pltpu.VMEM((2, chunk, d), jnp.bfloat16),
            pltpu.VMEM((2, chunk, d), jnp.bfloat16),
            pltpu.VMEM((N_DEV, chunk, d), jnp.bfloat16),
            pltpu.VMEM((chunk, d), jnp.float32),
            pltpu.SemaphoreType.DMA,
            pltpu.SemaphoreType.DMA((N_DEV - 1,)),
            pltpu.SemaphoreType.DMA((N_DEV - 1,)),
            pltpu.SemaphoreType.DMA((N_DEV - 1,)),
            pltpu.SemaphoreType.DMA((N_DEV - 1,)),
        ],
        compiler_params=pltpu.CompilerParams(
            collective_id=0,
            vmem_limit_bytes=60 * 1024 * 1024,
        ),
    )(partial_bf, resid, gamma2d)


# device time: 124175 ns/iter; 1.4981x vs baseline; 1.4981x over previous
import jax
import jax.numpy as jnp
from jax import lax
from jax.experimental import pallas as pl
from jax.experimental.pallas import tpu as pltpu

N_Y = 4
K = 8
HALF = 1024
CR = HALF // K


def kernel(partial, resid, gamma):
    _, m, d = partial.shape
    my_x_out = lax.axis_index("x")
    partial_bf = partial.reshape(m, d).astype(jnp.bfloat16)
    resid_bf = resid.astype(jnp.bfloat16)
    p_half = lax.dynamic_slice(partial_bf, (my_x_out * HALF, 0), (HALF, d))
    r_half = lax.dynamic_slice(resid_bf, (my_x_out * HALF, 0), (HALF, d))
    p_half = p_half.reshape(K, CR, d)
    r_half = r_half.reshape(K, CR, d)
    gamma2d = gamma.reshape(1, d)

    def body(p_ref, r_ref, g_ref, out_ref,
             fwd_in, rev_in, fwd_out, rev_out, norm_bf, x_in,
             fwd_in_sems, rev_in_sems, x_in_sems,
             fwd_out_sems, rev_out_sems, x_out_sems, exit_sem):
        my_x = lax.axis_index("x")
        my_y = lax.axis_index("y")
        my_z = lax.axis_index("z")
        f32 = jnp.float32

        def send_fwd(src_buf, c, to_y):
            pltpu.make_async_remote_copy(
                src_ref=src_buf.at[c], dst_ref=fwd_in.at[c],
                send_sem=fwd_out_sems.at[c], recv_sem=fwd_in_sems.at[c],
                device_id=(my_x, to_y, my_z),
                device_id_type=pl.DeviceIdType.MESH,
            ).start()

        def send_rev(src_buf, c, to_y):
            pltpu.make_async_remote_copy(
                src_ref=src_buf.at[c], dst_ref=rev_in.at[c],
                send_sem=rev_out_sems.at[c], recv_sem=rev_in_sems.at[c],
                device_id=(my_x, to_y, my_z),
                device_id_type=pl.DeviceIdType.MESH,
            ).start()

        def wait_in(buf, sems, c):
            pltpu.make_async_remote_copy(
                src_ref=buf.at[c], dst_ref=buf.at[c],
                send_sem=sems.at[c], recv_sem=sems.at[c],
                device_id=(my_x, my_y, my_z),
                device_id_type=pl.DeviceIdType.MESH,
            ).wait_recv()

        def wait_sent(src_buf, sems, c):
            pltpu.make_async_remote_copy(
                src_ref=src_buf.at[c], dst_ref=src_buf.at[c],
                send_sem=sems.at[c], recv_sem=sems.at[c],
                device_id=(my_x, my_y, my_z),
                device_id_type=pl.DeviceIdType.MESH,
            ).wait_send()

        def ln_store(c, total):
            yv = total + r_ref[c].astype(f32)
            ms = jnp.mean(yv * yv, axis=-1, keepdims=True)
            o = yv * lax.rsqrt(ms + 1e-6) * g_ref[...]
            out_ref[pl.ds(my_x * HALF + c * CR, CR), :] = o
            norm_bf[c] = o.astype(jnp.bfloat16)
            pltpu.make_async_remote_copy(
                src_ref=norm_bf.at[c], dst_ref=x_in.at[c],
                send_sem=x_out_sems.at[c], recv_sem=x_in_sems.at[c],
                device_id=(1 - my_x, my_y, my_z),
                device_id_type=pl.DeviceIdType.MESH,
            ).start()

        def nbr_ys(r):
            return [ny for ny in (r - 1, r + 1) if 0 <= ny < N_Y]

        barrier_sem = pltpu.get_barrier_semaphore()

        def entry_barrier(r):
            def _():
                for ny in nbr_ys(r):
                    pl.semaphore_signal(
                        barrier_sem, inc=1, device_id=(my_x, ny, my_z),
                        device_id_type=pl.DeviceIdType.MESH)
                pl.semaphore_signal(
                    barrier_sem, inc=1, device_id=(1 - my_x, r, my_z),
                    device_id_type=pl.DeviceIdType.MESH)
                pl.semaphore_wait(barrier_sem, len(nbr_ys(r)) + 1)
            return _

        for r in range(N_Y):
            pl.when(my_y == r)(entry_barrier(r))

        def role0():
            for c in range(K):
                send_fwd(p_ref, c, 1)
            for c in range(K):
                wait_in(rev_in, rev_in_sems, c)
                total = p_ref[c].astype(f32) + rev_in[c].astype(f32)
                ln_store(c, total)
            for c in range(K):
                wait_sent(p_ref, fwd_out_sems, c)

        def role1():
            for c in range(K):
                wait_in(fwd_in, fwd_in_sems, c)
                fwd_out[c] = (
                    fwd_in[c].astype(f32) + p_ref[c].astype(f32)
                ).astype(jnp.bfloat16)
                send_fwd(fwd_out, c, 2)
                wait_in(rev_in, rev_in_sems, c)
                rev_out[c] = (
                    rev_in[c].astype(f32) + p_ref[c].astype(f32)
                ).astype(jnp.bfloat16)
                send_rev(rev_out, c, 0)
                total = (fwd_in[c].astype(f32) + p_ref[c].astype(f32)
                         + rev_in[c].astype(f32))
                ln_store(c, total)
            for c in range(K):
                wait_sent(fwd_out, fwd_out_sems, c)
                wait_sent(rev_out, rev_out_sems, c)

        def role2():
            for c in range(K):
                wait_in(rev_in, rev_in_sems, c)
                rev_out[c] = (
                    rev_in[c].astype(f32) + p_ref[c].astype(f32)
                ).astype(jnp.bfloat16)
                send_rev(rev_out, c, 1)
                wait_in(fwd_in, fwd_in_sems, c)
                fwd_out[c] = (
                    fwd_in[c].astype(f32) + p_ref[c].astype(f32)
                ).astype(jnp.bfloat16)
                send_fwd(fwd_out, c, 3)
                total = (fwd_in[c].astype(f32) + p_ref[c].astype(f32)
                         + rev_in[c].astype(f32))
                ln_store(c, total)
            for c in range(K):
                wait_sent(rev_out, rev_out_sems, c)
                wait_sent(fwd_out, fwd_out_sems, c)

        def role3():
            for c in range(K):
                send_rev(p_ref, c, 2)
            for c in range(K):
                wait_in(fwd_in, fwd_in_sems, c)
                total = p_ref[c].astype(f32) + fwd_in[c].astype(f32)
                ln_store(c, total)
            for c in range(K):
                wait_sent(p_ref, rev_out_sems, c)

        pl.when(my_y == 0)(role0)
        pl.when(my_y == 1)(role1)
        pl.when(my_y == 2)(role2)
        pl.when(my_y == 3)(role3)

        for c in range(K):
            wait_in(x_in, x_in_sems, c)
            out_ref[pl.ds((1 - my_x) * HALF + c * CR, CR), :] = (
                x_in[c].astype(f32))
        for c in range(K):
            wait_sent(norm_bf, x_out_sems, c)

        def exit_barrier(r):
            def _():
                for ny in nbr_ys(r):
                    pl.semaphore_signal(
                        exit_sem, inc=1, device_id=(my_x, ny, my_z),
                        device_id_type=pl.DeviceIdType.MESH)
                pl.semaphore_signal(
                    exit_sem, inc=1, device_id=(1 - my_x, r, my_z),
                    device_id_type=pl.DeviceIdType.MESH)
                pl.semaphore_wait(exit_sem, len(nbr_ys(r)) + 1)
            return _

        for r in range(N_Y):
            pl.when(my_y == r)(exit_barrier(r))

    cdim = (K, CR, d)
    return pl.pallas_call(
        body,
        out_shape=jax.ShapeDtypeStruct((m, d), jnp.float32),
        in_specs=[
            pl.BlockSpec(memory_space=pltpu.VMEM),
            pl.BlockSpec(memory_space=pltpu.VMEM),
            pl.BlockSpec(memory_space=pltpu.VMEM),
        ],
        out_specs=pl.BlockSpec(memory_space=pltpu.VMEM),
        scratch_shapes=[
            pltpu.VMEM(cdim, jnp.bfloat16),
            pltpu.VMEM(cdim, jnp.bfloat16),
            pltpu.VMEM(cdim, jnp.bfloat16),
            pltpu.VMEM(cdim, jnp.bfloat16),
            pltpu.VMEM(cdim, jnp.bfloat16),
            pltpu.VMEM(cdim, jnp.bfloat16),
            pltpu.SemaphoreType.DMA((K,)),
            pltpu.SemaphoreType.DMA((K,)),
            pltpu.SemaphoreType.DMA((K,)),
            pltpu.SemaphoreType.DMA((K,)),
            pltpu.SemaphoreType.DMA((K,)),
            pltpu.SemaphoreType.DMA((K,)),
            pltpu.SemaphoreType.REGULAR,
        ],
        compiler_params=pltpu.CompilerParams(
            collective_id=0,
            vmem_limit_bytes=60 * 1024 * 1024,
        ),
    )(p_half, r_half, gamma2d)
